# baseline (speedup 1.0000x reference)
import functools

import jax
import jax.numpy as jnp
from jax import lax
from jax.experimental import pallas as pl
from jax.experimental.pallas import tpu as pltpu
from jax.experimental.pallas import tpu_sc as plsc

N = 10000
E = 320000
D = 128

_NC, _NS, _L = 2, 16, 16
_NW = _NC * _NS
_R_MAIN = 78
_R_TAIL = E // D - 31 * _R_MAIN
_R_CHUNK = 39
_N_CHUNKS = _R_MAIN // _R_CHUNK
_E_MAIN = _R_MAIN * D
_E_TAIL = _R_TAIL * D
_E_CHUNK = _R_CHUNK * D
_GROUPS_PER_ROW = D // _L

_SCORE_BN = 1024
_N_PAD = 10240


def _scores_body(x_ref, w_ref, b_ref, out_ref):
    i = pl.program_id(0)
    z = lax.dot_general(
        w_ref[...], x_ref[...],
        dimension_numbers=(((1,), (1,)), ((), ())),
        preferred_element_type=jnp.float32,
    )
    out_ref[pl.ds(i * _SCORE_BN, _SCORE_BN)] = jax.nn.sigmoid(z[0] + b_ref[0])


def _compute_scores(x, W, b):
    return pl.pallas_call(
        _scores_body,
        grid=(_N_PAD // _SCORE_BN,),
        out_shape=jax.ShapeDtypeStruct((_N_PAD,), jnp.float32),
        in_specs=[
            pl.BlockSpec((_SCORE_BN, D), lambda i: (i, 0), memory_space=pltpu.VMEM),
            pl.BlockSpec((1, D), lambda i: (0, 0), memory_space=pltpu.VMEM),
            pl.BlockSpec(memory_space=pltpu.SMEM),
        ],
        out_specs=pl.BlockSpec((_N_PAD,), lambda i: (0,), memory_space=pltpu.VMEM),
    )(x, W, b)


@functools.cache
def _build_edge_kernel():
    mesh = plsc.VectorSubcoreMesh(core_axis_name="c", subcore_axis_name="s")
    return pl.kernel(
        _edge_body,
        out_type=jax.ShapeDtypeStruct((E // D, 1, D), jnp.float32),
        mesh=mesh,
        scratch_types=[
            pltpu.VMEM((_N_PAD,), jnp.float32),
            pltpu.VMEM((2, _E_TAIL), jnp.int32),
            pltpu.VMEM((_R_TAIL, 1, D), jnp.float32),
            pltpu.SemaphoreType.DMA,
            pltpu.SemaphoreType.DMA,
            pltpu.SemaphoreType.DMA,
        ],
        compiler_params=pltpu.CompilerParams(needs_layout_passes=False),
    )


def _edge_body(scores_hbm, edge_hbm, out_hbm, scores_v, rc_v, out_v, sem, sem_t, sem_st):
    wid = lax.axis_index("s") * _NC + lax.axis_index("c")
    is_tail = wid == _NW - 1
    base = wid * _E_MAIN
    row_base = wid * _R_MAIN

    cp_s = pltpu.async_copy(scores_hbm, scores_v, sem)
    cp_idx = [
        pltpu.async_copy(
            edge_hbm.at[:, pl.ds(base + k * _E_CHUNK, _E_CHUNK)],
            rc_v.at[:, pl.ds(k * _E_CHUNK, _E_CHUNK)],
            sem,
        )
        for k in range(_N_CHUNKS)
    ]

    @pl.when(is_tail)
    def _():
        pltpu.async_copy(
            edge_hbm.at[:, pl.ds(32 * _E_MAIN, _E_TAIL - _E_MAIN)],
            rc_v.at[:, pl.ds(_E_MAIN, _E_TAIL - _E_MAIN)],
            sem_t,
        )

    def row_body(r):
        for j in range(_GROUPS_PER_ROW):
            off = r * D + j * _L
            rr = rc_v[0, pl.ds(off, _L)]
            cc = rc_v[1, pl.ds(off, _L)]
            sr = plsc.load_gather(scores_v, [rr])
            sc = plsc.load_gather(scores_v, [cc])
            out_v[r, 0, pl.ds(j * _L, _L)] = sr * sc

    cp_s.wait()
    cp_out = []
    for k in range(_N_CHUNKS):
        cp_idx[k].wait()
        plsc.parallel_loop(k * _R_CHUNK, (k + 1) * _R_CHUNK, 1)(row_body)
        cp_out.append(
            pltpu.async_copy(
                out_v.at[pl.ds(k * _R_CHUNK, _R_CHUNK), :, :],
                out_hbm.at[pl.ds(row_base + k * _R_CHUNK, _R_CHUNK), :, :],
                sem_st,
            )
        )

    @pl.when(is_tail)
    def _():
        pltpu.make_async_copy(
            edge_hbm.at[:, pl.ds(32 * _E_MAIN, _E_TAIL - _E_MAIN)],
            rc_v.at[:, pl.ds(_E_MAIN, _E_TAIL - _E_MAIN)],
            sem_t,
        ).wait()
        plsc.parallel_loop(_R_MAIN, _R_TAIL, 1)(row_body)
        pltpu.sync_copy(
            out_v.at[pl.ds(_R_MAIN, _R_TAIL - _R_MAIN), :, :],
            out_hbm.at[pl.ds(31 * _R_MAIN + _R_MAIN, _R_TAIL - _R_MAIN), :, :],
        )

    for cp in cp_out:
        cp.wait()


def kernel(x, edge_index, W, b):
    scores = _compute_scores(x, W, b)
    attn = _build_edge_kernel()(scores, edge_index.astype(jnp.int32))
    return attn.reshape(E, 1)


# speedup vs baseline: 1.1335x; 1.1335x over previous
import functools

import jax
import jax.numpy as jnp
from jax import lax
from jax.experimental import pallas as pl
from jax.experimental.pallas import tpu as pltpu
from jax.experimental.pallas import tpu_sc as plsc

N = 10000
E = 320000
D = 128

_NC, _NS, _L = 2, 16, 16
_NW = _NC * _NS
_R_MAIN = 78
_R_TAIL = E // D - 31 * _R_MAIN
_R_CHUNK = 39
_N_CHUNKS = _R_MAIN // _R_CHUNK
_E_MAIN = _R_MAIN * D
_E_TAIL = _R_TAIL * D
_E_CHUNK = _R_CHUNK * D
_GROUPS_PER_ROW = D // _L

def _scores_body(x_ref, w_ref, b_ref, out_ref):
    z = lax.dot_general(
        w_ref[...], x_ref[...],
        dimension_numbers=(((1,), (1,)), ((), ())),
        preferred_element_type=jnp.float32,
    )
    out_ref[...] = jax.nn.sigmoid(z[0] + b_ref[0])


def _compute_scores(x, W, b):
    return pl.pallas_call(
        _scores_body,
        out_shape=jax.ShapeDtypeStruct((N,), jnp.float32),
        in_specs=[
            pl.BlockSpec(memory_space=pltpu.VMEM),
            pl.BlockSpec(memory_space=pltpu.VMEM),
            pl.BlockSpec(memory_space=pltpu.SMEM),
        ],
        out_specs=pl.BlockSpec(memory_space=pltpu.VMEM),
    )(x, W, b)


@functools.cache
def _build_edge_kernel():
    mesh = plsc.VectorSubcoreMesh(core_axis_name="c", subcore_axis_name="s")
    return pl.kernel(
        _edge_body,
        out_type=jax.ShapeDtypeStruct((E // D, 1, D), jnp.float32),
        mesh=mesh,
        scratch_types=[
            pltpu.VMEM((N,), jnp.float32),
            pltpu.VMEM((2, _E_TAIL), jnp.int32),
            pltpu.VMEM((_R_TAIL, 1, D), jnp.float32),
            pltpu.SemaphoreType.DMA,
            pltpu.SemaphoreType.DMA,
            pltpu.SemaphoreType.DMA,
        ],
        compiler_params=pltpu.CompilerParams(needs_layout_passes=False),
    )


def _edge_body(scores_hbm, edge_hbm, out_hbm, scores_v, rc_v, out_v, sem, sem_t, sem_st):
    wid = lax.axis_index("s") * _NC + lax.axis_index("c")
    is_tail = wid == _NW - 1
    base = wid * _E_MAIN
    row_base = wid * _R_MAIN

    cp_s = pltpu.async_copy(scores_hbm, scores_v, sem)
    cp_idx = [
        pltpu.async_copy(
            edge_hbm.at[:, pl.ds(base + k * _E_CHUNK, _E_CHUNK)],
            rc_v.at[:, pl.ds(k * _E_CHUNK, _E_CHUNK)],
            sem,
        )
        for k in range(_N_CHUNKS)
    ]

    @pl.when(is_tail)
    def _():
        pltpu.async_copy(
            edge_hbm.at[:, pl.ds(32 * _E_MAIN, _E_TAIL - _E_MAIN)],
            rc_v.at[:, pl.ds(_E_MAIN, _E_TAIL - _E_MAIN)],
            sem_t,
        )

    def row_body(r):
        for j in range(_GROUPS_PER_ROW):
            off = r * D + j * _L
            rr = rc_v[0, pl.ds(off, _L)]
            cc = rc_v[1, pl.ds(off, _L)]
            sr = plsc.load_gather(scores_v, [rr])
            sc = plsc.load_gather(scores_v, [cc])
            out_v[r, 0, pl.ds(j * _L, _L)] = sr * sc

    cp_s.wait()
    cp_out = []
    for k in range(_N_CHUNKS):
        cp_idx[k].wait()
        plsc.parallel_loop(k * _R_CHUNK, (k + 1) * _R_CHUNK, 1)(row_body)
        cp_out.append(
            pltpu.async_copy(
                out_v.at[pl.ds(k * _R_CHUNK, _R_CHUNK), :, :],
                out_hbm.at[pl.ds(row_base + k * _R_CHUNK, _R_CHUNK), :, :],
                sem_st,
            )
        )

    @pl.when(is_tail)
    def _():
        pltpu.make_async_copy(
            edge_hbm.at[:, pl.ds(32 * _E_MAIN, _E_TAIL - _E_MAIN)],
            rc_v.at[:, pl.ds(_E_MAIN, _E_TAIL - _E_MAIN)],
            sem_t,
        ).wait()
        plsc.parallel_loop(_R_MAIN, _R_TAIL, 1)(row_body)
        pltpu.sync_copy(
            out_v.at[pl.ds(_R_MAIN, _R_TAIL - _R_MAIN), :, :],
            out_hbm.at[pl.ds(31 * _R_MAIN + _R_MAIN, _R_TAIL - _R_MAIN), :, :],
        )

    for cp in cp_out:
        cp.wait()


def kernel(x, edge_index, W, b):
    scores = _compute_scores(x, W, b)
    attn = _build_edge_kernel()(scores, edge_index.astype(jnp.int32))
    return attn.reshape(E, 1)


# speedup vs baseline: 1.1514x; 1.0158x over previous
import functools

import jax
import jax.numpy as jnp
from jax import lax
from jax.experimental import pallas as pl
from jax.experimental.pallas import tpu as pltpu
from jax.experimental.pallas import tpu_sc as plsc

N = 10000
E = 320000
D = 128

_NC, _NS, _L = 2, 16, 16
_NW = _NC * _NS
_R_MAIN = 78
_R_TAIL = E // D - 31 * _R_MAIN
_R_CHUNK = 39
_N_CHUNKS = _R_MAIN // _R_CHUNK
_E_MAIN = _R_MAIN * D
_E_TAIL = _R_TAIL * D
_E_CHUNK = _R_CHUNK * D
_GROUPS_PER_ROW = D // _L

def _scores_body(x_ref, w_ref, b_ref, out_ref):
    z = lax.dot_general(
        w_ref[...], x_ref[...],
        dimension_numbers=(((1,), (1,)), ((), ())),
        preferred_element_type=jnp.float32,
    )
    out_ref[...] = jax.nn.sigmoid(z[0] + b_ref[0])


def _compute_scores(x, W, b):
    return pl.pallas_call(
        _scores_body,
        out_shape=jax.ShapeDtypeStruct((N,), jnp.float32),
        in_specs=[
            pl.BlockSpec(memory_space=pltpu.VMEM),
            pl.BlockSpec(memory_space=pltpu.VMEM),
            pl.BlockSpec(memory_space=pltpu.SMEM),
        ],
        out_specs=pl.BlockSpec(memory_space=pltpu.VMEM),
    )(x, W, b)


@functools.cache
def _build_edge_kernel():
    mesh = plsc.VectorSubcoreMesh(core_axis_name="c", subcore_axis_name="s")
    return pl.kernel(
        _edge_body,
        out_type=jax.ShapeDtypeStruct((E // D, 1, D), jnp.float32),
        mesh=mesh,
        scratch_types=[
            pltpu.VMEM((N,), jnp.float32),
            pltpu.VMEM((2, _E_TAIL), jnp.int32),
            pltpu.VMEM((_R_TAIL, 1, D), jnp.float32),
            pltpu.SemaphoreType.DMA,
            pltpu.SemaphoreType.DMA,
            pltpu.SemaphoreType.DMA,
        ],
        compiler_params=pltpu.CompilerParams(needs_layout_passes=False),
    )


def _edge_body(scores_hbm, edge_hbm, out_hbm, scores_v, rc_v, out_v, sem, sem_t, sem_st):
    wid = lax.axis_index("s") * _NC + lax.axis_index("c")
    is_tail = wid == _NW - 1
    base = wid * _E_MAIN
    row_base = wid * _R_MAIN

    cp_s = pltpu.async_copy(scores_hbm, scores_v, sem)
    cp_idx = [
        pltpu.async_copy(
            edge_hbm.at[:, pl.ds(base + k * _E_CHUNK, _E_CHUNK)],
            rc_v.at[:, pl.ds(k * _E_CHUNK, _E_CHUNK)],
            sem,
        )
        for k in range(_N_CHUNKS)
    ]

    @pl.when(is_tail)
    def _():
        pltpu.async_copy(
            edge_hbm.at[:, pl.ds(32 * _E_MAIN, _E_TAIL - _E_MAIN)],
            rc_v.at[:, pl.ds(_E_MAIN, _E_TAIL - _E_MAIN)],
            sem_t,
        )

    def row_body(r):
        idx = []
        for j in range(_GROUPS_PER_ROW):
            off = r * D + j * _L
            idx.append((rc_v[0, pl.ds(off, _L)], rc_v[1, pl.ds(off, _L)]))
        gath = [
            (plsc.load_gather(scores_v, [rr]), plsc.load_gather(scores_v, [cc]))
            for rr, cc in idx
        ]
        for j, (sr, sc) in enumerate(gath):
            out_v[r, 0, pl.ds(j * _L, _L)] = sr * sc

    cp_s.wait()
    cp_out = []
    for k in range(_N_CHUNKS):
        cp_idx[k].wait()
        plsc.parallel_loop(k * _R_CHUNK, (k + 1) * _R_CHUNK, 1)(row_body)
        cp_out.append(
            pltpu.async_copy(
                out_v.at[pl.ds(k * _R_CHUNK, _R_CHUNK), :, :],
                out_hbm.at[pl.ds(row_base + k * _R_CHUNK, _R_CHUNK), :, :],
                sem_st,
            )
        )

    @pl.when(is_tail)
    def _():
        pltpu.make_async_copy(
            edge_hbm.at[:, pl.ds(32 * _E_MAIN, _E_TAIL - _E_MAIN)],
            rc_v.at[:, pl.ds(_E_MAIN, _E_TAIL - _E_MAIN)],
            sem_t,
        ).wait()
        plsc.parallel_loop(_R_MAIN, _R_TAIL, 1)(row_body)
        pltpu.sync_copy(
            out_v.at[pl.ds(_R_MAIN, _R_TAIL - _R_MAIN), :, :],
            out_hbm.at[pl.ds(31 * _R_MAIN + _R_MAIN, _R_TAIL - _R_MAIN), :, :],
        )

    for cp in cp_out:
        cp.wait()


def kernel(x, edge_index, W, b):
    scores = _compute_scores(x, W, b)
    attn = _build_edge_kernel()(scores, edge_index.astype(jnp.int32))
    return attn.reshape(E, 1)
